# baseline (speedup 1.0000x reference)
import functools

import jax
import jax.numpy as jnp
from jax import lax
from jax.experimental import pallas as pl
from jax.experimental.pallas import tpu as pltpu
from jax.experimental.pallas import tpu_sc as plsc

B = 16384
NCAT = 26
V = 100000
D = 16
NCONT = 13
NBIN = 16
H1 = 128
H2 = 64
H2P = 128
CBW = 32
EMBW = NCAT * D

try:
    _info = plsc.get_sparse_core_info()
    _NC = _info.num_cores
    _NS = _info.num_subcores
except Exception:
    _NC, _NS = 2, 16
NW = _NC * _NS
ROWS = B * NCAT
ROWS_PER_W = ROWS // NW
N_CHUNKS = 8
CH = ROWS_PER_W // N_CHUNKS


VP = 100096
_NLT = VP // 128
_NFULL = _NLT - 1
_VTAIL = V - 128 * _NFULL
_TROWS = NCAT * VP * D // 128
_NTILES = NCAT * _NLT
_FROWS = VP * D // 128


def _sc_detile(emb_t, emb_tail):
    mesh = plsc.VectorSubcoreMesh(core_axis_name="c", subcore_axis_name="s")

    @functools.partial(
        pl.kernel,
        mesh=mesh,
        compiler_params=pltpu.CompilerParams(use_tc_tiling_on_sc=True,
                                             needs_layout_passes=False),
        out_type=jax.ShapeDtypeStruct((_TROWS, 128), jnp.float32),
        scratch_types=[
            pltpu.VMEM((D, 128), jnp.float32),
            pltpu.VMEM((D, _VTAIL), jnp.float32),
            pltpu.VMEM((D, 128), jnp.float32),
        ],
    )
    def detile_k(in_hbm, tail_hbm, out_hbm, inbuf, tailbuf, outbuf):
        wid = lax.axis_index("s") * _NC + lax.axis_index("c")
        start = wid * _NTILES // NW
        end = (wid + 1) * _NTILES // NW
        jrows = lax.iota(jnp.int32, D)

        def transpose_tile(src, nv):
            for r in range(nv // 8):
                for vlo in range(8):
                    col = jnp.full((D,), 8 * r + vlo, jnp.int32)
                    vec = plsc.load_gather(src, [jrows, col])
                    outbuf[r, pl.ds(vlo * D, D)] = vec

        def body(p, carry):
            f = p // _NLT
            lt = p % _NLT
            row_in = pl.multiple_of(f * D, 8)
            row_out = pl.multiple_of(f * _FROWS + lt * D, 8)

            @pl.when(lt < _NFULL)
            def _full():
                col_in = pl.multiple_of(lt * 128, 128)
                pltpu.sync_copy(
                    in_hbm.at[pl.ds(row_in, D), pl.ds(col_in, 128)], inbuf)
                transpose_tile(inbuf, 128)

            @pl.when(lt == _NFULL)
            def _tail():
                pltpu.sync_copy(tail_hbm.at[pl.ds(row_in, D)], tailbuf)
                transpose_tile(tailbuf, _VTAIL)

            pltpu.sync_copy(outbuf, out_hbm.at[pl.ds(row_out, D)])
            return carry

        lax.fori_loop(start, end, body, 0)

    return detile_k(emb_t, emb_tail)


def _sc_gather(table, idx):
    mesh = plsc.VectorSubcoreMesh(core_axis_name="c", subcore_axis_name="s")

    @functools.partial(
        pl.kernel,
        mesh=mesh,
        compiler_params=pltpu.CompilerParams(use_tc_tiling_on_sc=False),
        out_type=jax.ShapeDtypeStruct((ROWS, D), jnp.float32),
        scratch_types=[
            pltpu.VMEM((ROWS_PER_W,), jnp.int32),
            pltpu.VMEM((CH, D), jnp.float32),
            pltpu.VMEM((CH, D), jnp.float32),
            pltpu.SemaphoreType.DMA,
            pltpu.SemaphoreType.DMA,
        ],
    )
    def gather_k(table_hbm, idx_hbm, out_hbm, idx_v, buf0, buf1, sem0, sem1):
        wid = lax.axis_index("s") * _NC + lax.axis_index("c")
        base = wid * ROWS_PER_W
        pltpu.sync_copy(idx_hbm.at[pl.ds(base, ROWS_PER_W)], idx_v)
        bufs = (buf0, buf1)
        sems = (sem0, sem1)

        def issue(c):
            return pltpu.async_copy(
                table_hbm.at[idx_v.at[pl.ds(c * CH, CH)]], bufs[c % 2], sems[c % 2]
            )

        cps = [None, None]
        cps[0] = issue(0)
        for c in range(N_CHUNKS):
            if c + 1 < N_CHUNKS:
                cps[(c + 1) % 2] = issue(c + 1)
            cps[c % 2].wait()
            pltpu.sync_copy(bufs[c % 2], out_hbm.at[pl.ds(base + c * CH, CH)])

    return gather_k(table, idx)


BT = 2048


def _mlp_body(xg_ref, cb_ref, w1e_ref, w1cb_ref, b1_ref, w2_ref, b2_ref,
              w3_ref, out_ref):
    h = jnp.dot(xg_ref[...], w1e_ref[...], preferred_element_type=jnp.float32)
    h = h + jnp.dot(cb_ref[...], w1cb_ref[...],
                    preferred_element_type=jnp.float32)
    h = jnp.maximum(h + b1_ref[...], 0.0)
    h2 = jnp.dot(h, w2_ref[...], preferred_element_type=jnp.float32)
    h2 = jnp.maximum(h2 + b2_ref[...], 0.0)
    out_ref[...] = jnp.sum(h2 * w3_ref[...], axis=1)


def _tc_mlp(xg, cb, w1e, w1cb, b1f, w2f, b2f, w3f):
    grid = (B // BT,)
    return pl.pallas_call(
        _mlp_body,
        grid=grid,
        in_specs=[
            pl.BlockSpec((BT, EMBW), lambda i: (i, 0)),
            pl.BlockSpec((BT, CBW), lambda i: (i, 0)),
            pl.BlockSpec((EMBW, H1), lambda i: (0, 0)),
            pl.BlockSpec((CBW, H1), lambda i: (0, 0)),
            pl.BlockSpec((1, H1), lambda i: (0, 0)),
            pl.BlockSpec((H1, H2P), lambda i: (0, 0)),
            pl.BlockSpec((1, H2P), lambda i: (0, 0)),
            pl.BlockSpec((1, H2P), lambda i: (0, 0)),
        ],
        out_specs=pl.BlockSpec((BT,), lambda i: (i,)),
        out_shape=jax.ShapeDtypeStruct((B,), jnp.float32),
    )(xg, cb, w1e, w1cb, b1f, w2f, b2f, w3f)


def kernel(categorical, continuous, binary, emb, W1, b1, g1, be1,
           W2, b2, g2, be2, W3, b3):
    eps = 1e-5
    inv = 1.0 / jnp.sqrt(1.0 + eps)
    s1 = g1 * inv
    w1f = W1 * s1[:, None]
    b1f = b1 * s1 + be1
    s2 = g2 * inv
    w2f = W2 * s2[:, None]
    b2f = b2 * s2 + be2

    ncb = NCONT + NBIN
    w1cb = jnp.zeros((CBW, H1), jnp.float32).at[:ncb].set(w1f[:, :ncb].T)
    w1e = w1f[:, ncb:].T
    cb = jnp.concatenate(
        [continuous, binary,
         jnp.zeros((B, CBW - ncb), jnp.float32)], axis=1)

    w2p = jnp.zeros((H1, H2P), jnp.float32).at[:, :H2].set(w2f.T)
    b2p = jnp.zeros((1, H2P), jnp.float32).at[0, :H2].set(b2f)
    w3p = jnp.zeros((1, H2P), jnp.float32).at[0, :H2].set(W3[0])

    idx = (categorical + (jnp.arange(NCAT, dtype=jnp.int32) * VP)[None, :])
    idx = idx.reshape(-1)
    emb_t = jnp.swapaxes(emb, 1, 2).reshape(NCAT * D, V)
    emb_tail = lax.slice(emb_t, (0, 128 * _NFULL), (NCAT * D, V))
    table = _sc_detile(emb_t, emb_tail).reshape(NCAT * VP, D)

    xg = _sc_gather(table, idx).reshape(B, EMBW)

    out = _tc_mlp(xg, cb, w1e, w1cb, b1f.reshape(1, H1), w2p, b2p, w3p)
    return out + b3[0]


# speedup vs baseline: 2.0902x; 2.0902x over previous
import functools

import jax
import jax.numpy as jnp
from jax import lax
from jax.experimental import pallas as pl
from jax.experimental.pallas import tpu as pltpu
from jax.experimental.pallas import tpu_sc as plsc

B = 16384
NCAT = 26
V = 100000
D = 16
NCONT = 13
NBIN = 16
H1 = 128
H2 = 64
H2P = 128
CBW = 32
EMBW = NCAT * D

try:
    _info = plsc.get_sparse_core_info()
    _NC = _info.num_cores
    _NS = _info.num_subcores
except Exception:
    _NC, _NS = 2, 16
NW = _NC * _NS
ROWS = B * NCAT
ROWS_PER_W = ROWS // NW
N_CHUNKS = 8
CH = ROWS_PER_W // N_CHUNKS


VP = 100096
_NLT = VP // 128
_NFULL = _NLT - 1
_VTAIL = V - 128 * _NFULL
_TROWS = NCAT * VP * D // 128
_FROWS = VP * D // 128
_UK = 11
_UPF = _NFULL // _UK
_NUNITS = NCAT * _UPF
_UPW = 58


def _sc_detile(emb_t, emb_tail):
    mesh = plsc.VectorSubcoreMesh(core_axis_name="c", subcore_axis_name="s")

    @functools.partial(
        pl.kernel,
        mesh=mesh,
        compiler_params=pltpu.CompilerParams(use_tc_tiling_on_sc=True,
                                             needs_layout_passes=False),
        out_type=jax.ShapeDtypeStruct((_TROWS, 128), jnp.float32),
        scratch_types=[
            pltpu.VMEM((D, 128 * _UK), jnp.float32),
            pltpu.VMEM((D, 128 * _UK), jnp.float32),
            pltpu.VMEM((D * _UK, 128), jnp.float32),
            pltpu.VMEM((D * _UK, 128), jnp.float32),
            pltpu.VMEM((D, _VTAIL), jnp.float32),
            pltpu.SemaphoreType.DMA,
            pltpu.SemaphoreType.DMA,
        ],
    )
    def detile_k(in_hbm, tail_hbm, out_hbm, in0, in1, out0, out1, tailbuf,
                 si0, si1):
        wid = lax.axis_index("s") * _NC + lax.axis_index("c")
        ustart = wid * _UPW
        ulimit = jnp.minimum(_NUNITS, ustart + _UPW)
        ins = (in0, in1)
        outs = (out0, out1)
        sems = (si0, si1)
        jrows = lax.iota(jnp.int32, D)

        def in_slices(u):
            f = u // _UPF
            g = u % _UPF
            row = pl.multiple_of(f * D, 8)
            col = pl.multiple_of(g * (128 * _UK), 128)
            return in_hbm.at[pl.ds(row, D), pl.ds(col, 128 * _UK)]

        def start_in(u, b):
            @pl.when(u < ulimit)
            def _():
                pltpu.async_copy(in_slices(u), ins[b], sems[b])

        def wait_in(u, b):
            @pl.when(u < ulimit)
            def _():
                pltpu.make_async_copy(in_slices(u), ins[b], sems[b]).wait()

        def transpose_rows(src, dst, t, nrows):
            for r in range(nrows):
                vecs = []
                for vlo in range(8):
                    col = jnp.zeros((D,), jnp.int32) + (t * 128 + 8 * r + vlo)
                    vecs.append(plsc.load_gather(src, [jrows, col]))
                for vlo in range(8):
                    dst[t * D + r, pl.ds(vlo * D, D)] = vecs[vlo]

        def compute(u, b):
            @pl.when(u < ulimit)
            def _():
                f = u // _UPF
                g = u % _UPF

                def tbody(t, c):
                    transpose_rows(ins[b], outs[b], t, D)
                    return c

                lax.fori_loop(0, _UK, tbody, 0)
                orow = pl.multiple_of(f * _FROWS + g * (D * _UK), 8)
                pltpu.sync_copy(outs[b], out_hbm.at[pl.ds(orow, D * _UK)])

        start_in(ustart, 0)

        def outer(i, c):
            u0 = ustart + 2 * i
            wait_in(u0, 0)
            start_in(u0 + 1, 1)
            compute(u0, 0)
            wait_in(u0 + 1, 1)
            start_in(u0 + 2, 0)
            compute(u0 + 1, 1)
            return c

        lax.fori_loop(0, _UPW // 2, outer, 0)

        @pl.when(wid < NCAT)
        def _tail():
            f = wid
            pltpu.sync_copy(tail_hbm.at[pl.ds(pl.multiple_of(f * D, 8), D)],
                            tailbuf)
            for r in range(_VTAIL // 8):
                vecs = []
                for vlo in range(8):
                    col = jnp.full((D,), 8 * r + vlo, jnp.int32)
                    vecs.append(plsc.load_gather(tailbuf, [jrows, col]))
                for vlo in range(8):
                    out0[r, pl.ds(vlo * D, D)] = vecs[vlo]
            orow = pl.multiple_of(f * _FROWS + _UPF * _UK * D, 8)
            pltpu.sync_copy(out0.at[pl.ds(0, D)], out_hbm.at[pl.ds(orow, D)])

    return detile_k(emb_t, emb_tail)


def _sc_gather(table, idx):
    mesh = plsc.VectorSubcoreMesh(core_axis_name="c", subcore_axis_name="s")

    @functools.partial(
        pl.kernel,
        mesh=mesh,
        compiler_params=pltpu.CompilerParams(use_tc_tiling_on_sc=False),
        out_type=jax.ShapeDtypeStruct((ROWS, D), jnp.float32),
        scratch_types=[
            pltpu.VMEM((ROWS_PER_W,), jnp.int32),
            pltpu.VMEM((CH, D), jnp.float32),
            pltpu.VMEM((CH, D), jnp.float32),
            pltpu.SemaphoreType.DMA,
            pltpu.SemaphoreType.DMA,
        ],
    )
    def gather_k(table_hbm, idx_hbm, out_hbm, idx_v, buf0, buf1, sem0, sem1):
        wid = lax.axis_index("s") * _NC + lax.axis_index("c")
        base = wid * ROWS_PER_W
        pltpu.sync_copy(idx_hbm.at[pl.ds(base, ROWS_PER_W)], idx_v)
        bufs = (buf0, buf1)
        sems = (sem0, sem1)

        def issue(c):
            return pltpu.async_copy(
                table_hbm.at[idx_v.at[pl.ds(c * CH, CH)]], bufs[c % 2], sems[c % 2]
            )

        cps = [None, None]
        cps[0] = issue(0)
        for c in range(N_CHUNKS):
            if c + 1 < N_CHUNKS:
                cps[(c + 1) % 2] = issue(c + 1)
            cps[c % 2].wait()
            pltpu.sync_copy(bufs[c % 2], out_hbm.at[pl.ds(base + c * CH, CH)])

    return gather_k(table, idx)


BT = 2048


def _mlp_body(xg_ref, cb_ref, w1e_ref, w1cb_ref, b1_ref, w2_ref, b2_ref,
              w3_ref, out_ref):
    h = jnp.dot(xg_ref[...], w1e_ref[...], preferred_element_type=jnp.float32)
    h = h + jnp.dot(cb_ref[...], w1cb_ref[...],
                    preferred_element_type=jnp.float32)
    h = jnp.maximum(h + b1_ref[...], 0.0)
    h2 = jnp.dot(h, w2_ref[...], preferred_element_type=jnp.float32)
    h2 = jnp.maximum(h2 + b2_ref[...], 0.0)
    out_ref[...] = jnp.sum(h2 * w3_ref[...], axis=1)


def _tc_mlp(xg, cb, w1e, w1cb, b1f, w2f, b2f, w3f):
    grid = (B // BT,)
    return pl.pallas_call(
        _mlp_body,
        grid=grid,
        in_specs=[
            pl.BlockSpec((BT, EMBW), lambda i: (i, 0)),
            pl.BlockSpec((BT, CBW), lambda i: (i, 0)),
            pl.BlockSpec((EMBW, H1), lambda i: (0, 0)),
            pl.BlockSpec((CBW, H1), lambda i: (0, 0)),
            pl.BlockSpec((1, H1), lambda i: (0, 0)),
            pl.BlockSpec((H1, H2P), lambda i: (0, 0)),
            pl.BlockSpec((1, H2P), lambda i: (0, 0)),
            pl.BlockSpec((1, H2P), lambda i: (0, 0)),
        ],
        out_specs=pl.BlockSpec((BT,), lambda i: (i,)),
        out_shape=jax.ShapeDtypeStruct((B,), jnp.float32),
    )(xg, cb, w1e, w1cb, b1f, w2f, b2f, w3f)


def kernel(categorical, continuous, binary, emb, W1, b1, g1, be1,
           W2, b2, g2, be2, W3, b3):
    eps = 1e-5
    inv = 1.0 / jnp.sqrt(1.0 + eps)
    s1 = g1 * inv
    w1f = W1 * s1[:, None]
    b1f = b1 * s1 + be1
    s2 = g2 * inv
    w2f = W2 * s2[:, None]
    b2f = b2 * s2 + be2

    ncb = NCONT + NBIN
    w1cb = jnp.zeros((CBW, H1), jnp.float32).at[:ncb].set(w1f[:, :ncb].T)
    w1e = w1f[:, ncb:].T
    cb = jnp.concatenate(
        [continuous, binary,
         jnp.zeros((B, CBW - ncb), jnp.float32)], axis=1)

    w2p = jnp.zeros((H1, H2P), jnp.float32).at[:, :H2].set(w2f.T)
    b2p = jnp.zeros((1, H2P), jnp.float32).at[0, :H2].set(b2f)
    w3p = jnp.zeros((1, H2P), jnp.float32).at[0, :H2].set(W3[0])

    idx = (categorical + (jnp.arange(NCAT, dtype=jnp.int32) * VP)[None, :])
    idx = idx.reshape(-1)
    emb_t = jnp.swapaxes(emb, 1, 2).reshape(NCAT * D, V)
    emb_tail = lax.slice(emb_t, (0, 128 * _NFULL), (NCAT * D, V))
    table = _sc_detile(emb_t, emb_tail).reshape(NCAT * VP, D)

    xg = _sc_gather(table, idx).reshape(B, EMBW)

    out = _tc_mlp(xg, cb, w1e, w1cb, b1f.reshape(1, H1), w2p, b2p, w3p)
    return out + b3[0]


# speedup vs baseline: 2.1453x; 1.0264x over previous
import functools

import jax
import jax.numpy as jnp
from jax import lax
from jax.experimental import pallas as pl
from jax.experimental.pallas import tpu as pltpu
from jax.experimental.pallas import tpu_sc as plsc

B = 16384
NCAT = 26
V = 100000
D = 16
NCONT = 13
NBIN = 16
H1 = 128
H2 = 64
H2P = 128
CBW = 32
EMBW = NCAT * D

try:
    _info = plsc.get_sparse_core_info()
    _NC = _info.num_cores
    _NS = _info.num_subcores
except Exception:
    _NC, _NS = 2, 16
NW = _NC * _NS
ROWS = B * NCAT
ROWS_PER_W = ROWS // NW
N_CHUNKS = 8
CH = ROWS_PER_W // N_CHUNKS


VP = 100096
_NLT = VP // 128
_NFULL = _NLT - 1
_VTAIL = V - 128 * _NFULL
_TROWS = NCAT * VP * D // 128
_FROWS = VP * D // 128
_UK = 11
_UPF = _NFULL // _UK
_NUNITS = NCAT * _UPF
_UPW = 58


def _sc_detile(emb_t, emb_tail):
    mesh = plsc.VectorSubcoreMesh(core_axis_name="c", subcore_axis_name="s")

    @functools.partial(
        pl.kernel,
        mesh=mesh,
        compiler_params=pltpu.CompilerParams(use_tc_tiling_on_sc=True,
                                             needs_layout_passes=False),
        out_type=jax.ShapeDtypeStruct((_TROWS, 128), jnp.float32),
        scratch_types=[
            pltpu.VMEM((D, 128 * _UK), jnp.float32),
            pltpu.VMEM((D, 128 * _UK), jnp.float32),
            pltpu.VMEM((D * _UK, 128), jnp.float32),
            pltpu.VMEM((D * _UK, 128), jnp.float32),
            pltpu.VMEM((D, _VTAIL), jnp.float32),
            pltpu.SemaphoreType.DMA,
            pltpu.SemaphoreType.DMA,
        ],
    )
    def detile_k(in_hbm, tail_hbm, out_hbm, in0, in1, out0, out1, tailbuf,
                 si0, si1):
        wid = lax.axis_index("s") * _NC + lax.axis_index("c")
        ustart = wid * _UPW
        ulimit = jnp.minimum(_NUNITS, ustart + _UPW)
        ins = (in0, in1)
        outs = (out0, out1)
        sems = (si0, si1)
        jrows = lax.iota(jnp.int32, D)

        def in_slices(u):
            f = u // _UPF
            g = u % _UPF
            row = pl.multiple_of(f * D, 8)
            col = pl.multiple_of(g * (128 * _UK), 128)
            return in_hbm.at[pl.ds(row, D), pl.ds(col, 128 * _UK)]

        def start_in(u, b):
            @pl.when(u < ulimit)
            def _():
                pltpu.async_copy(in_slices(u), ins[b], sems[b])

        def wait_in(u, b):
            @pl.when(u < ulimit)
            def _():
                pltpu.make_async_copy(in_slices(u), ins[b], sems[b]).wait()

        def transpose_rows(src, dst, t, nrows):
            for r in range(nrows):
                vecs = []
                for vlo in range(8):
                    col = jnp.zeros((D,), jnp.int32) + (t * 128 + 8 * r + vlo)
                    vecs.append(plsc.load_gather(src, [jrows, col]))
                for vlo in range(8):
                    dst[t * D + r, pl.ds(vlo * D, D)] = vecs[vlo]

        def compute(u, b):
            @pl.when(u < ulimit)
            def _():
                f = u // _UPF
                g = u % _UPF

                @plsc.parallel_loop(0, _UK, unroll=4)
                def _tiles(t):
                    transpose_rows(ins[b], outs[b], t, D)
                orow = pl.multiple_of(f * _FROWS + g * (D * _UK), 8)
                pltpu.sync_copy(outs[b], out_hbm.at[pl.ds(orow, D * _UK)])

        start_in(ustart, 0)

        def outer(i, c):
            u0 = ustart + 2 * i
            wait_in(u0, 0)
            start_in(u0 + 1, 1)
            compute(u0, 0)
            wait_in(u0 + 1, 1)
            start_in(u0 + 2, 0)
            compute(u0 + 1, 1)
            return c

        lax.fori_loop(0, _UPW // 2, outer, 0)

        @pl.when(wid < NCAT)
        def _tail():
            f = wid
            pltpu.sync_copy(tail_hbm.at[pl.ds(pl.multiple_of(f * D, 8), D)],
                            tailbuf)
            for r in range(_VTAIL // 8):
                vecs = []
                for vlo in range(8):
                    col = jnp.full((D,), 8 * r + vlo, jnp.int32)
                    vecs.append(plsc.load_gather(tailbuf, [jrows, col]))
                for vlo in range(8):
                    out0[r, pl.ds(vlo * D, D)] = vecs[vlo]
            orow = pl.multiple_of(f * _FROWS + _UPF * _UK * D, 8)
            pltpu.sync_copy(out0.at[pl.ds(0, D)], out_hbm.at[pl.ds(orow, D)])

    return detile_k(emb_t, emb_tail)


def _sc_gather(table, idx):
    mesh = plsc.VectorSubcoreMesh(core_axis_name="c", subcore_axis_name="s")

    @functools.partial(
        pl.kernel,
        mesh=mesh,
        compiler_params=pltpu.CompilerParams(use_tc_tiling_on_sc=False),
        out_type=jax.ShapeDtypeStruct((ROWS, D), jnp.float32),
        scratch_types=[
            pltpu.VMEM((ROWS_PER_W,), jnp.int32),
            pltpu.VMEM((CH, D), jnp.float32),
            pltpu.VMEM((CH, D), jnp.float32),
            pltpu.SemaphoreType.DMA,
            pltpu.SemaphoreType.DMA,
        ],
    )
    def gather_k(table_hbm, idx_hbm, out_hbm, idx_v, buf0, buf1, sem0, sem1):
        wid = lax.axis_index("s") * _NC + lax.axis_index("c")
        base = wid * ROWS_PER_W
        pltpu.sync_copy(idx_hbm.at[pl.ds(base, ROWS_PER_W)], idx_v)
        bufs = (buf0, buf1)
        sems = (sem0, sem1)

        def issue(c):
            return pltpu.async_copy(
                table_hbm.at[idx_v.at[pl.ds(c * CH, CH)]], bufs[c % 2], sems[c % 2]
            )

        cps = [None, None]
        cps[0] = issue(0)
        for c in range(N_CHUNKS):
            if c + 1 < N_CHUNKS:
                cps[(c + 1) % 2] = issue(c + 1)
            cps[c % 2].wait()
            pltpu.sync_copy(bufs[c % 2], out_hbm.at[pl.ds(base + c * CH, CH)])

    return gather_k(table, idx)


BT = 2048


def _mlp_body(xg_ref, cb_ref, w1e_ref, w1cb_ref, b1_ref, w2_ref, b2_ref,
              w3_ref, out_ref):
    h = jnp.dot(xg_ref[...], w1e_ref[...], preferred_element_type=jnp.float32)
    h = h + jnp.dot(cb_ref[...], w1cb_ref[...],
                    preferred_element_type=jnp.float32)
    h = jnp.maximum(h + b1_ref[...], 0.0)
    h2 = jnp.dot(h, w2_ref[...], preferred_element_type=jnp.float32)
    h2 = jnp.maximum(h2 + b2_ref[...], 0.0)
    out_ref[...] = jnp.sum(h2 * w3_ref[...], axis=1)


def _tc_mlp(xg, cb, w1e, w1cb, b1f, w2f, b2f, w3f):
    grid = (B // BT,)
    return pl.pallas_call(
        _mlp_body,
        grid=grid,
        in_specs=[
            pl.BlockSpec((BT, EMBW), lambda i: (i, 0)),
            pl.BlockSpec((BT, CBW), lambda i: (i, 0)),
            pl.BlockSpec((EMBW, H1), lambda i: (0, 0)),
            pl.BlockSpec((CBW, H1), lambda i: (0, 0)),
            pl.BlockSpec((1, H1), lambda i: (0, 0)),
            pl.BlockSpec((H1, H2P), lambda i: (0, 0)),
            pl.BlockSpec((1, H2P), lambda i: (0, 0)),
            pl.BlockSpec((1, H2P), lambda i: (0, 0)),
        ],
        out_specs=pl.BlockSpec((BT,), lambda i: (i,)),
        out_shape=jax.ShapeDtypeStruct((B,), jnp.float32),
    )(xg, cb, w1e, w1cb, b1f, w2f, b2f, w3f)


def kernel(categorical, continuous, binary, emb, W1, b1, g1, be1,
           W2, b2, g2, be2, W3, b3):
    eps = 1e-5
    inv = 1.0 / jnp.sqrt(1.0 + eps)
    s1 = g1 * inv
    w1f = W1 * s1[:, None]
    b1f = b1 * s1 + be1
    s2 = g2 * inv
    w2f = W2 * s2[:, None]
    b2f = b2 * s2 + be2

    ncb = NCONT + NBIN
    w1cb = jnp.zeros((CBW, H1), jnp.float32).at[:ncb].set(w1f[:, :ncb].T)
    w1e = w1f[:, ncb:].T
    cb = jnp.concatenate(
        [continuous, binary,
         jnp.zeros((B, CBW - ncb), jnp.float32)], axis=1)

    w2p = jnp.zeros((H1, H2P), jnp.float32).at[:, :H2].set(w2f.T)
    b2p = jnp.zeros((1, H2P), jnp.float32).at[0, :H2].set(b2f)
    w3p = jnp.zeros((1, H2P), jnp.float32).at[0, :H2].set(W3[0])

    idx = (categorical + (jnp.arange(NCAT, dtype=jnp.int32) * VP)[None, :])
    idx = idx.reshape(-1)
    emb_t = jnp.swapaxes(emb, 1, 2).reshape(NCAT * D, V)
    emb_tail = lax.slice(emb_t, (0, 128 * _NFULL), (NCAT * D, V))
    table = _sc_detile(emb_t, emb_tail).reshape(NCAT * VP, D)

    xg = _sc_gather(table, idx).reshape(B, EMBW)

    out = _tc_mlp(xg, cb, w1e, w1cb, b1f.reshape(1, H1), w2p, b2p, w3p)
    return out + b3[0]


# speedup vs baseline: 2.1469x; 1.0007x over previous
import functools

import jax
import jax.numpy as jnp
from jax import lax
from jax.experimental import pallas as pl
from jax.experimental.pallas import tpu as pltpu
from jax.experimental.pallas import tpu_sc as plsc

B = 16384
NCAT = 26
V = 100000
D = 16
NCONT = 13
NBIN = 16
H1 = 128
H2 = 64
H2P = 128
CBW = 32
EMBW = NCAT * D

try:
    _info = plsc.get_sparse_core_info()
    _NC = _info.num_cores
    _NS = _info.num_subcores
except Exception:
    _NC, _NS = 2, 16
NW = _NC * _NS
ROWS = B * NCAT
ROWS_PER_W = ROWS // NW
N_CHUNKS = 8
CH = ROWS_PER_W // N_CHUNKS


VP = 100096
_NLT = VP // 128
_NFULL = _NLT - 1
_VTAIL = V - 128 * _NFULL
_TROWS = NCAT * VP * D // 128
_FROWS = VP * D // 128
_UK = 11
_UPF = _NFULL // _UK
_NUNITS = NCAT * _UPF
_UPW = 58


def _sc_detile(emb_t, emb_tail):
    mesh = plsc.VectorSubcoreMesh(core_axis_name="c", subcore_axis_name="s")

    @functools.partial(
        pl.kernel,
        mesh=mesh,
        compiler_params=pltpu.CompilerParams(use_tc_tiling_on_sc=True,
                                             needs_layout_passes=False),
        out_type=jax.ShapeDtypeStruct((_TROWS, 128), jnp.float32),
        scratch_types=[
            pltpu.VMEM((D, 128 * _UK + 1), jnp.float32),
            pltpu.VMEM((D, 128 * _UK + 1), jnp.float32),
            pltpu.VMEM((D * _UK, 128), jnp.float32),
            pltpu.VMEM((D * _UK, 128), jnp.float32),
            pltpu.VMEM((D, _VTAIL), jnp.float32),
            pltpu.SemaphoreType.DMA,
            pltpu.SemaphoreType.DMA,
        ],
    )
    def detile_k(in_hbm, tail_hbm, out_hbm, in0, in1, out0, out1, tailbuf,
                 si0, si1):
        wid = lax.axis_index("s") * _NC + lax.axis_index("c")
        ustart = wid * _UPW
        ulimit = jnp.minimum(_NUNITS, ustart + _UPW)
        ins = (in0, in1)
        outs = (out0, out1)
        sems = (si0, si1)
        jrows = lax.iota(jnp.int32, D)

        def in_slices(u):
            f = u // _UPF
            g = u % _UPF
            row = pl.multiple_of(f * D, 8)
            col = pl.multiple_of(g * (128 * _UK), 128)
            return in_hbm.at[pl.ds(row, D), pl.ds(col, 128 * _UK)]

        def start_in(u, b):
            @pl.when(u < ulimit)
            def _():
                pltpu.async_copy(in_slices(u),
                                 ins[b].at[:, pl.ds(0, 128 * _UK)], sems[b])

        def wait_in(u, b):
            @pl.when(u < ulimit)
            def _():
                pltpu.make_async_copy(in_slices(u),
                                      ins[b].at[:, pl.ds(0, 128 * _UK)],
                                      sems[b]).wait()

        def transpose_rows(src, dst, t, nrows):
            for r in range(nrows):
                vecs = []
                for vlo in range(8):
                    col = jnp.zeros((D,), jnp.int32) + (t * 128 + 8 * r + vlo)
                    vecs.append(plsc.load_gather(src, [jrows, col]))
                for vlo in range(8):
                    dst[t * D + r, pl.ds(vlo * D, D)] = vecs[vlo]

        def compute(u, b):
            @pl.when(u < ulimit)
            def _():
                f = u // _UPF
                g = u % _UPF

                @plsc.parallel_loop(0, _UK, unroll=4)
                def _tiles(t):
                    transpose_rows(ins[b], outs[b], t, D)
                orow = pl.multiple_of(f * _FROWS + g * (D * _UK), 8)
                pltpu.sync_copy(outs[b], out_hbm.at[pl.ds(orow, D * _UK)])

        start_in(ustart, 0)

        def outer(i, c):
            u0 = ustart + 2 * i
            wait_in(u0, 0)
            start_in(u0 + 1, 1)
            compute(u0, 0)
            wait_in(u0 + 1, 1)
            start_in(u0 + 2, 0)
            compute(u0 + 1, 1)
            return c

        lax.fori_loop(0, _UPW // 2, outer, 0)

        @pl.when(wid < NCAT)
        def _tail():
            f = wid
            pltpu.sync_copy(tail_hbm.at[pl.ds(pl.multiple_of(f * D, 8), D)],
                            tailbuf)
            for r in range(_VTAIL // 8):
                vecs = []
                for vlo in range(8):
                    col = jnp.full((D,), 8 * r + vlo, jnp.int32)
                    vecs.append(plsc.load_gather(tailbuf, [jrows, col]))
                for vlo in range(8):
                    out0[r, pl.ds(vlo * D, D)] = vecs[vlo]
            orow = pl.multiple_of(f * _FROWS + _UPF * _UK * D, 8)
            pltpu.sync_copy(out0.at[pl.ds(0, D)], out_hbm.at[pl.ds(orow, D)])

    return detile_k(emb_t, emb_tail)


def _sc_gather(table, idx):
    mesh = plsc.VectorSubcoreMesh(core_axis_name="c", subcore_axis_name="s")

    @functools.partial(
        pl.kernel,
        mesh=mesh,
        compiler_params=pltpu.CompilerParams(use_tc_tiling_on_sc=False),
        out_type=jax.ShapeDtypeStruct((ROWS, D), jnp.float32),
        scratch_types=[
            pltpu.VMEM((ROWS_PER_W,), jnp.int32),
            pltpu.VMEM((CH, D), jnp.float32),
            pltpu.VMEM((CH, D), jnp.float32),
            pltpu.SemaphoreType.DMA,
            pltpu.SemaphoreType.DMA,
        ],
    )
    def gather_k(table_hbm, idx_hbm, out_hbm, idx_v, buf0, buf1, sem0, sem1):
        wid = lax.axis_index("s") * _NC + lax.axis_index("c")
        base = wid * ROWS_PER_W
        pltpu.sync_copy(idx_hbm.at[pl.ds(base, ROWS_PER_W)], idx_v)
        bufs = (buf0, buf1)
        sems = (sem0, sem1)

        def issue(c):
            return pltpu.async_copy(
                table_hbm.at[idx_v.at[pl.ds(c * CH, CH)]], bufs[c % 2], sems[c % 2]
            )

        cps = [None, None]
        cps[0] = issue(0)
        for c in range(N_CHUNKS):
            if c + 1 < N_CHUNKS:
                cps[(c + 1) % 2] = issue(c + 1)
            cps[c % 2].wait()
            pltpu.sync_copy(bufs[c % 2], out_hbm.at[pl.ds(base + c * CH, CH)])

    return gather_k(table, idx)


BT = 2048


def _mlp_body(xg_ref, cb_ref, w1e_ref, w1cb_ref, b1_ref, w2_ref, b2_ref,
              w3_ref, out_ref):
    h = jnp.dot(xg_ref[...], w1e_ref[...], preferred_element_type=jnp.float32)
    h = h + jnp.dot(cb_ref[...], w1cb_ref[...],
                    preferred_element_type=jnp.float32)
    h = jnp.maximum(h + b1_ref[...], 0.0)
    h2 = jnp.dot(h, w2_ref[...], preferred_element_type=jnp.float32)
    h2 = jnp.maximum(h2 + b2_ref[...], 0.0)
    out_ref[...] = jnp.sum(h2 * w3_ref[...], axis=1)


def _tc_mlp(xg, cb, w1e, w1cb, b1f, w2f, b2f, w3f):
    grid = (B // BT,)
    return pl.pallas_call(
        _mlp_body,
        grid=grid,
        in_specs=[
            pl.BlockSpec((BT, EMBW), lambda i: (i, 0)),
            pl.BlockSpec((BT, CBW), lambda i: (i, 0)),
            pl.BlockSpec((EMBW, H1), lambda i: (0, 0)),
            pl.BlockSpec((CBW, H1), lambda i: (0, 0)),
            pl.BlockSpec((1, H1), lambda i: (0, 0)),
            pl.BlockSpec((H1, H2P), lambda i: (0, 0)),
            pl.BlockSpec((1, H2P), lambda i: (0, 0)),
            pl.BlockSpec((1, H2P), lambda i: (0, 0)),
        ],
        out_specs=pl.BlockSpec((BT,), lambda i: (i,)),
        out_shape=jax.ShapeDtypeStruct((B,), jnp.float32),
    )(xg, cb, w1e, w1cb, b1f, w2f, b2f, w3f)


def kernel(categorical, continuous, binary, emb, W1, b1, g1, be1,
           W2, b2, g2, be2, W3, b3):
    eps = 1e-5
    inv = 1.0 / jnp.sqrt(1.0 + eps)
    s1 = g1 * inv
    w1f = W1 * s1[:, None]
    b1f = b1 * s1 + be1
    s2 = g2 * inv
    w2f = W2 * s2[:, None]
    b2f = b2 * s2 + be2

    ncb = NCONT + NBIN
    w1cb = jnp.zeros((CBW, H1), jnp.float32).at[:ncb].set(w1f[:, :ncb].T)
    w1e = w1f[:, ncb:].T
    cb = jnp.concatenate(
        [continuous, binary,
         jnp.zeros((B, CBW - ncb), jnp.float32)], axis=1)

    w2p = jnp.zeros((H1, H2P), jnp.float32).at[:, :H2].set(w2f.T)
    b2p = jnp.zeros((1, H2P), jnp.float32).at[0, :H2].set(b2f)
    w3p = jnp.zeros((1, H2P), jnp.float32).at[0, :H2].set(W3[0])

    idx = (categorical + (jnp.arange(NCAT, dtype=jnp.int32) * VP)[None, :])
    idx = idx.reshape(-1)
    emb_t = jnp.swapaxes(emb, 1, 2).reshape(NCAT * D, V)
    emb_tail = lax.slice(emb_t, (0, 128 * _NFULL), (NCAT * D, V))
    table = _sc_detile(emb_t, emb_tail).reshape(NCAT * VP, D)

    xg = _sc_gather(table, idx).reshape(B, EMBW)

    out = _tc_mlp(xg, cb, w1e, w1cb, b1f.reshape(1, H1), w2p, b2p, w3p)
    return out + b3[0]


# speedup vs baseline: 3.1542x; 1.4692x over previous
import functools

import jax
import jax.numpy as jnp
from jax import lax
from jax.experimental import pallas as pl
from jax.experimental.pallas import tpu as pltpu
from jax.experimental.pallas import tpu_sc as plsc

B = 16384
NCAT = 26
V = 100000
D = 16
NCONT = 13
NBIN = 16
H1 = 128
H2 = 64
H2P = 128
CBW = 32
EMBW = NCAT * D

try:
    _info = plsc.get_sparse_core_info()
    _NC = _info.num_cores
    _NS = _info.num_subcores
except Exception:
    _NC, _NS = 2, 16
NW = _NC * _NS
ROWS = B * NCAT
ROWS_PER_W = ROWS // NW
N_CHUNKS = 8
CH = ROWS_PER_W // N_CHUNKS


VP = 100096
_NLT = VP // 128
_NFULL = _NLT - 1
_VTAIL = V - 128 * _NFULL
_TROWS = NCAT * VP * D // 128
_FROWS = VP * D // 128
_UK = 11
_UPF = _NFULL // _UK
_NUNITS = NCAT * _UPF
_UPW = 58


def _sc_detile(emb_t, emb_tail):
    mesh = plsc.VectorSubcoreMesh(core_axis_name="c", subcore_axis_name="s")

    @functools.partial(
        pl.kernel,
        mesh=mesh,
        compiler_params=pltpu.CompilerParams(use_tc_tiling_on_sc=True,
                                             needs_layout_passes=False),
        out_type=jax.ShapeDtypeStruct((_TROWS, 128), jnp.float32),
        scratch_types=[
            pltpu.VMEM((D, 128 * _UK + 1), jnp.float32),
            pltpu.VMEM((D, 128 * _UK + 1), jnp.float32),
            pltpu.VMEM((D * _UK, 128), jnp.float32),
            pltpu.VMEM((D * _UK, 128), jnp.float32),
            pltpu.VMEM((D, _VTAIL), jnp.float32),
            pltpu.SemaphoreType.DMA,
            pltpu.SemaphoreType.DMA,
        ],
    )
    def detile_k(in_hbm, tail_hbm, out_hbm, in0, in1, out0, out1, tailbuf,
                 si0, si1):
        wid = lax.axis_index("s") * _NC + lax.axis_index("c")
        ustart = wid * _UPW
        ulimit = jnp.minimum(_NUNITS, ustart + _UPW)
        ins = (in0, in1)
        outs = (out0, out1)
        sems = (si0, si1)
        jrows = lax.iota(jnp.int32, D)

        def in_slices(u):
            f = u // _UPF
            g = u % _UPF
            row = pl.multiple_of(f * D, 8)
            col = pl.multiple_of(g * (128 * _UK), 128)
            return in_hbm.at[pl.ds(row, D), pl.ds(col, 128 * _UK)]

        def start_in(u, b):
            @pl.when(u < ulimit)
            def _():
                pltpu.async_copy(in_slices(u),
                                 ins[b].at[:, pl.ds(0, 128 * _UK)], sems[b])

        def wait_in(u, b):
            @pl.when(u < ulimit)
            def _():
                pltpu.make_async_copy(in_slices(u),
                                      ins[b].at[:, pl.ds(0, 128 * _UK)],
                                      sems[b]).wait()

        def transpose_rows(src, dst, t, nrows):
            del nrows
            for w in range(8):
                for s in range(D):
                    off = (jrows + s) % D
                    lcol = off + (t * 128 + D * w)
                    val = plsc.load_gather(src, [jrows, lcol])
                    srow = (off // 8) + (t * D + 2 * w)
                    scol = (off % 8) * D + jrows
                    plsc.store_scatter(dst, [srow, scol], val)

        def compute(u, b):
            @pl.when(u < ulimit)
            def _():
                f = u // _UPF
                g = u % _UPF

                @plsc.parallel_loop(0, _UK, unroll=4)
                def _tiles(t):
                    transpose_rows(ins[b], outs[b], t, D)
                orow = pl.multiple_of(f * _FROWS + g * (D * _UK), 8)
                pltpu.sync_copy(outs[b], out_hbm.at[pl.ds(orow, D * _UK)])

        start_in(ustart, 0)

        def outer(i, c):
            u0 = ustart + 2 * i
            wait_in(u0, 0)
            start_in(u0 + 1, 1)
            compute(u0, 0)
            wait_in(u0 + 1, 1)
            start_in(u0 + 2, 0)
            compute(u0 + 1, 1)
            return c

        lax.fori_loop(0, _UPW // 2, outer, 0)

        @pl.when(wid < NCAT)
        def _tail():
            f = wid
            pltpu.sync_copy(tail_hbm.at[pl.ds(pl.multiple_of(f * D, 8), D)],
                            tailbuf)
            for r in range(_VTAIL // 8):
                vecs = []
                for vlo in range(8):
                    col = jnp.full((D,), 8 * r + vlo, jnp.int32)
                    vecs.append(plsc.load_gather(tailbuf, [jrows, col]))
                for vlo in range(8):
                    out0[r, pl.ds(vlo * D, D)] = vecs[vlo]
            orow = pl.multiple_of(f * _FROWS + _UPF * _UK * D, 8)
            pltpu.sync_copy(out0.at[pl.ds(0, D)], out_hbm.at[pl.ds(orow, D)])

    return detile_k(emb_t, emb_tail)


def _sc_gather(table, idx):
    mesh = plsc.VectorSubcoreMesh(core_axis_name="c", subcore_axis_name="s")

    @functools.partial(
        pl.kernel,
        mesh=mesh,
        compiler_params=pltpu.CompilerParams(use_tc_tiling_on_sc=False),
        out_type=jax.ShapeDtypeStruct((ROWS, D), jnp.float32),
        scratch_types=[
            pltpu.VMEM((ROWS_PER_W,), jnp.int32),
            pltpu.VMEM((CH, D), jnp.float32),
            pltpu.VMEM((CH, D), jnp.float32),
            pltpu.SemaphoreType.DMA,
            pltpu.SemaphoreType.DMA,
        ],
    )
    def gather_k(table_hbm, idx_hbm, out_hbm, idx_v, buf0, buf1, sem0, sem1):
        wid = lax.axis_index("s") * _NC + lax.axis_index("c")
        base = wid * ROWS_PER_W
        pltpu.sync_copy(idx_hbm.at[pl.ds(base, ROWS_PER_W)], idx_v)
        bufs = (buf0, buf1)
        sems = (sem0, sem1)

        def issue(c):
            return pltpu.async_copy(
                table_hbm.at[idx_v.at[pl.ds(c * CH, CH)]], bufs[c % 2], sems[c % 2]
            )

        cps = [None, None]
        cps[0] = issue(0)
        for c in range(N_CHUNKS):
            if c + 1 < N_CHUNKS:
                cps[(c + 1) % 2] = issue(c + 1)
            cps[c % 2].wait()
            pltpu.sync_copy(bufs[c % 2], out_hbm.at[pl.ds(base + c * CH, CH)])

    return gather_k(table, idx)


BT = 2048


def _mlp_body(xg_ref, cb_ref, w1e_ref, w1cb_ref, b1_ref, w2_ref, b2_ref,
              w3_ref, out_ref):
    h = jnp.dot(xg_ref[...], w1e_ref[...], preferred_element_type=jnp.float32)
    h = h + jnp.dot(cb_ref[...], w1cb_ref[...],
                    preferred_element_type=jnp.float32)
    h = jnp.maximum(h + b1_ref[...], 0.0)
    h2 = jnp.dot(h, w2_ref[...], preferred_element_type=jnp.float32)
    h2 = jnp.maximum(h2 + b2_ref[...], 0.0)
    out_ref[...] = jnp.sum(h2 * w3_ref[...], axis=1)


def _tc_mlp(xg, cb, w1e, w1cb, b1f, w2f, b2f, w3f):
    grid = (B // BT,)
    return pl.pallas_call(
        _mlp_body,
        grid=grid,
        in_specs=[
            pl.BlockSpec((BT, EMBW), lambda i: (i, 0)),
            pl.BlockSpec((BT, CBW), lambda i: (i, 0)),
            pl.BlockSpec((EMBW, H1), lambda i: (0, 0)),
            pl.BlockSpec((CBW, H1), lambda i: (0, 0)),
            pl.BlockSpec((1, H1), lambda i: (0, 0)),
            pl.BlockSpec((H1, H2P), lambda i: (0, 0)),
            pl.BlockSpec((1, H2P), lambda i: (0, 0)),
            pl.BlockSpec((1, H2P), lambda i: (0, 0)),
        ],
        out_specs=pl.BlockSpec((BT,), lambda i: (i,)),
        out_shape=jax.ShapeDtypeStruct((B,), jnp.float32),
    )(xg, cb, w1e, w1cb, b1f, w2f, b2f, w3f)


def kernel(categorical, continuous, binary, emb, W1, b1, g1, be1,
           W2, b2, g2, be2, W3, b3):
    eps = 1e-5
    inv = 1.0 / jnp.sqrt(1.0 + eps)
    s1 = g1 * inv
    w1f = W1 * s1[:, None]
    b1f = b1 * s1 + be1
    s2 = g2 * inv
    w2f = W2 * s2[:, None]
    b2f = b2 * s2 + be2

    ncb = NCONT + NBIN
    w1cb = jnp.zeros((CBW, H1), jnp.float32).at[:ncb].set(w1f[:, :ncb].T)
    w1e = w1f[:, ncb:].T
    cb = jnp.concatenate(
        [continuous, binary,
         jnp.zeros((B, CBW - ncb), jnp.float32)], axis=1)

    w2p = jnp.zeros((H1, H2P), jnp.float32).at[:, :H2].set(w2f.T)
    b2p = jnp.zeros((1, H2P), jnp.float32).at[0, :H2].set(b2f)
    w3p = jnp.zeros((1, H2P), jnp.float32).at[0, :H2].set(W3[0])

    idx = (categorical + (jnp.arange(NCAT, dtype=jnp.int32) * VP)[None, :])
    idx = idx.reshape(-1)
    emb_t = jnp.swapaxes(emb, 1, 2).reshape(NCAT * D, V)
    emb_tail = lax.slice(emb_t, (0, 128 * _NFULL), (NCAT * D, V))
    table = _sc_detile(emb_t, emb_tail).reshape(NCAT * VP, D)

    xg = _sc_gather(table, idx).reshape(B, EMBW)

    out = _tc_mlp(xg, cb, w1e, w1cb, b1f.reshape(1, H1), w2p, b2p, w3p)
    return out + b3[0]


# speedup vs baseline: 3.1687x; 1.0046x over previous
import functools

import jax
import jax.numpy as jnp
from jax import lax
from jax.experimental import pallas as pl
from jax.experimental.pallas import tpu as pltpu
from jax.experimental.pallas import tpu_sc as plsc

B = 16384
NCAT = 26
V = 100000
D = 16
NCONT = 13
NBIN = 16
H1 = 128
H2 = 64
H2P = 128
CBW = 32
EMBW = NCAT * D

try:
    _info = plsc.get_sparse_core_info()
    _NC = _info.num_cores
    _NS = _info.num_subcores
except Exception:
    _NC, _NS = 2, 16
NW = _NC * _NS
ROWS = B * NCAT
ROWS_PER_W = ROWS // NW
N_CHUNKS = 8
CH = ROWS_PER_W // N_CHUNKS


VP = 100096
_NLT = VP // 128
_NFULL = _NLT - 1
_VTAIL = V - 128 * _NFULL
_TROWS = NCAT * VP * D // 128
_FROWS = VP * D // 128
_UK = 11
_UPF = _NFULL // _UK
_NUNITS = NCAT * _UPF
_UPW = 58


def _sc_detile(emb_t, emb_tail):
    mesh = plsc.VectorSubcoreMesh(core_axis_name="c", subcore_axis_name="s")

    @functools.partial(
        pl.kernel,
        mesh=mesh,
        compiler_params=pltpu.CompilerParams(use_tc_tiling_on_sc=True,
                                             needs_layout_passes=False),
        out_type=jax.ShapeDtypeStruct((_TROWS, 128), jnp.float32),
        scratch_types=[
            pltpu.VMEM((D, 128 * _UK + 1), jnp.float32),
            pltpu.VMEM((D, 128 * _UK + 1), jnp.float32),
            pltpu.VMEM((D * _UK, 128), jnp.float32),
            pltpu.VMEM((D * _UK, 128), jnp.float32),
            pltpu.VMEM((D, _VTAIL), jnp.float32),
            pltpu.SemaphoreType.DMA,
            pltpu.SemaphoreType.DMA,
        ],
    )
    def detile_k(in_hbm, tail_hbm, out_hbm, in0, in1, out0, out1, tailbuf,
                 si0, si1):
        wid = lax.axis_index("s") * _NC + lax.axis_index("c")
        ustart = wid * _UPW
        ulimit = jnp.minimum(_NUNITS, ustart + _UPW)
        ins = (in0, in1)
        outs = (out0, out1)
        sems = (si0, si1)
        jrows = lax.iota(jnp.int32, D)

        def out_slice(u):
            f = u // _UPF
            g = u % _UPF
            orow = pl.multiple_of(f * _FROWS + g * (D * _UK), 8)
            return out_hbm.at[pl.ds(orow, D * _UK)]

        def in_slices(u):
            f = u // _UPF
            g = u % _UPF
            row = pl.multiple_of(f * D, 8)
            col = pl.multiple_of(g * (128 * _UK), 128)
            return in_hbm.at[pl.ds(row, D), pl.ds(col, 128 * _UK)]

        def start_in(u, b):
            @pl.when(u < ulimit)
            def _():
                pltpu.async_copy(in_slices(u),
                                 ins[b].at[:, pl.ds(0, 128 * _UK)], sems[b])

        def wait_in(u, b):
            @pl.when(u < ulimit)
            def _():
                pltpu.make_async_copy(in_slices(u),
                                      ins[b].at[:, pl.ds(0, 128 * _UK)],
                                      sems[b]).wait()

        def transpose_rows(src, dst, t, nrows):
            del nrows
            for w in range(8):
                for s in range(D):
                    off = (jrows + s) % D
                    lcol = off + (t * 128 + D * w)
                    val = plsc.load_gather(src, [jrows, lcol])
                    srow = (off // 8) + (t * D + 2 * w)
                    scol = (off % 8) * D + jrows
                    plsc.store_scatter(dst, [srow, scol], val)

        def compute(u, b):
            @pl.when(u < ulimit)
            def _():
                @plsc.parallel_loop(0, _UK, unroll=4)
                def _tiles(t):
                    transpose_rows(ins[b], outs[b], t, D)

                pltpu.sync_copy(outs[b], out_slice(u))

        start_in(ustart, 0)

        def outer(i, c):
            u0 = ustart + 2 * i
            wait_in(u0, 0)
            start_in(u0 + 1, 1)
            compute(u0, 0)
            wait_in(u0 + 1, 1)
            start_in(u0 + 2, 0)
            compute(u0 + 1, 1)
            return c

        lax.fori_loop(0, _UPW // 2, outer, 0)

        @pl.when(wid < NCAT)
        def _tail():
            f = wid
            pltpu.sync_copy(tail_hbm.at[pl.ds(pl.multiple_of(f * D, 8), D)],
                            tailbuf)
            for r in range(_VTAIL // 8):
                vecs = []
                for vlo in range(8):
                    col = jnp.full((D,), 8 * r + vlo, jnp.int32)
                    vecs.append(plsc.load_gather(tailbuf, [jrows, col]))
                for vlo in range(8):
                    out0[r, pl.ds(vlo * D, D)] = vecs[vlo]
            orow = pl.multiple_of(f * _FROWS + _UPF * _UK * D, 8)
            pltpu.sync_copy(out0.at[pl.ds(0, D)], out_hbm.at[pl.ds(orow, D)])

    return detile_k(emb_t, emb_tail)


def _sc_gather(table, idx):
    mesh = plsc.VectorSubcoreMesh(core_axis_name="c", subcore_axis_name="s")

    @functools.partial(
        pl.kernel,
        mesh=mesh,
        compiler_params=pltpu.CompilerParams(use_tc_tiling_on_sc=False),
        out_type=jax.ShapeDtypeStruct((ROWS, D), jnp.float32),
        scratch_types=[
            pltpu.VMEM((ROWS_PER_W,), jnp.int32),
            pltpu.VMEM((CH, D), jnp.float32),
            pltpu.VMEM((CH, D), jnp.float32),
            pltpu.SemaphoreType.DMA,
            pltpu.SemaphoreType.DMA,
        ],
    )
    def gather_k(table_hbm, idx_hbm, out_hbm, idx_v, buf0, buf1, sem0, sem1):
        wid = lax.axis_index("s") * _NC + lax.axis_index("c")
        base = wid * ROWS_PER_W
        pltpu.sync_copy(idx_hbm.at[pl.ds(base, ROWS_PER_W)], idx_v)
        bufs = (buf0, buf1)
        sems = (sem0, sem1)

        def issue(c):
            return pltpu.async_copy(
                table_hbm.at[idx_v.at[pl.ds(c * CH, CH)]], bufs[c % 2], sems[c % 2]
            )

        cps = [None, None]
        cps[0] = issue(0)
        for c in range(N_CHUNKS):
            if c + 1 < N_CHUNKS:
                cps[(c + 1) % 2] = issue(c + 1)
            cps[c % 2].wait()
            pltpu.sync_copy(bufs[c % 2], out_hbm.at[pl.ds(base + c * CH, CH)])

    return gather_k(table, idx)


BT = 2048


def _mlp_body(xg_ref, cb_ref, w1e_ref, w1cb_ref, b1_ref, w2_ref, b2_ref,
              w3_ref, out_ref):
    h = jnp.dot(xg_ref[...], w1e_ref[...], preferred_element_type=jnp.float32)
    h = h + jnp.dot(cb_ref[...], w1cb_ref[...],
                    preferred_element_type=jnp.float32)
    h = jnp.maximum(h + b1_ref[...], 0.0)
    h2 = jnp.dot(h, w2_ref[...], preferred_element_type=jnp.float32)
    h2 = jnp.maximum(h2 + b2_ref[...], 0.0)
    out_ref[...] = jnp.sum(h2 * w3_ref[...], axis=1)


def _tc_mlp(xg, cb, w1e, w1cb, b1f, w2f, b2f, w3f):
    grid = (B // BT,)
    return pl.pallas_call(
        _mlp_body,
        grid=grid,
        in_specs=[
            pl.BlockSpec((BT, EMBW), lambda i: (i, 0)),
            pl.BlockSpec((BT, CBW), lambda i: (i, 0)),
            pl.BlockSpec((EMBW, H1), lambda i: (0, 0)),
            pl.BlockSpec((CBW, H1), lambda i: (0, 0)),
            pl.BlockSpec((1, H1), lambda i: (0, 0)),
            pl.BlockSpec((H1, H2P), lambda i: (0, 0)),
            pl.BlockSpec((1, H2P), lambda i: (0, 0)),
            pl.BlockSpec((1, H2P), lambda i: (0, 0)),
        ],
        out_specs=pl.BlockSpec((BT,), lambda i: (i,)),
        out_shape=jax.ShapeDtypeStruct((B,), jnp.float32),
    )(xg, cb, w1e, w1cb, b1f, w2f, b2f, w3f)


def kernel(categorical, continuous, binary, emb, W1, b1, g1, be1,
           W2, b2, g2, be2, W3, b3):
    eps = 1e-5
    inv = 1.0 / jnp.sqrt(1.0 + eps)
    s1 = g1 * inv
    w1f = W1 * s1[:, None]
    b1f = b1 * s1 + be1
    s2 = g2 * inv
    w2f = W2 * s2[:, None]
    b2f = b2 * s2 + be2

    ncb = NCONT + NBIN
    w1cb = jnp.zeros((CBW, H1), jnp.float32).at[:ncb].set(w1f[:, :ncb].T)
    w1e = w1f[:, ncb:].T
    cb = jnp.concatenate(
        [continuous, binary,
         jnp.zeros((B, CBW - ncb), jnp.float32)], axis=1)

    w2p = jnp.zeros((H1, H2P), jnp.float32).at[:, :H2].set(w2f.T)
    b2p = jnp.zeros((1, H2P), jnp.float32).at[0, :H2].set(b2f)
    w3p = jnp.zeros((1, H2P), jnp.float32).at[0, :H2].set(W3[0])

    idx = (categorical + (jnp.arange(NCAT, dtype=jnp.int32) * VP)[None, :])
    idx = idx.reshape(-1)
    emb_t = jnp.swapaxes(emb, 1, 2).reshape(NCAT * D, V)
    emb_tail = lax.slice(emb_t, (0, 128 * _NFULL), (NCAT * D, V))
    table = _sc_detile(emb_t, emb_tail).reshape(NCAT * VP, D)

    xg = _sc_gather(table, idx).reshape(B, EMBW)

    out = _tc_mlp(xg, cb, w1e, w1cb, b1f.reshape(1, H1), w2p, b2p, w3p)
    return out + b3[0]
